# baseline (speedup 1.0000x reference)
import functools

import jax
import jax.numpy as jnp
from jax import lax
from jax.experimental import pallas as pl
from jax.experimental.pallas import tpu as pltpu
from jax.experimental.pallas import tpu_sc as plsc

_B = 16384
_VOCAB = 1_000_000
_NS = 26
_NH = 50
_ND = 13
_NF = _NS + _NH
_NT = _NS + 1
_L = 16

_NC = 2
_NSUB = 16
_NW = _NC * _NSUB
_BPW = _B // _NW
_GPW = _BPW // _L
_NIDX = _NF * _BPW
_CHUNK = 128
_NCHUNK = _NIDX // _CHUNK


def _body(*refs):
    sp_hbm, vl_hbm, dn_hbm, wb_hbm, tab_hbm, out_hbm = refs[:6]
    idxs, vals, dense, wb, accs, ssem, gsem = refs[6:]

    wid = lax.axis_index("s") * _NC + lax.axis_index("c")
    base = wid * _BPW

    def sp_stage(f, c):
        pltpu.make_async_copy(
            sp_hbm.at[pl.ds(f * _B + base, _BPW)],
            idxs.at[pl.ds(f * _BPW, _BPW)], ssem).start()
        return c

    def vl_stage(h, c):
        pltpu.make_async_copy(
            vl_hbm.at[pl.ds(h * _B + base, _BPW)],
            idxs.at[pl.ds((_NS + h) * _BPW, _BPW)], ssem).start()
        return c

    def dn_stage(d, c):
        pltpu.make_async_copy(
            dn_hbm.at[pl.ds(d * _B + base, _BPW)],
            dense.at[pl.ds(d * _BPW, _BPW)], ssem).start()
        return c

    lax.fori_loop(0, _NS, sp_stage, 0)
    lax.fori_loop(0, _NH, vl_stage, 0)
    lax.fori_loop(0, _ND, dn_stage, 0)
    pltpu.sync_copy(wb_hbm, wb)

    def stage_drain(f, c):
        pltpu.make_async_copy(
            sp_hbm.at[pl.ds(base, _BPW)],
            idxs.at[pl.ds(0, _BPW)], ssem).wait()
        return c

    lax.fori_loop(0, _NF + _ND, stage_drain, 0)

    def gfire(r, c):
        pltpu.make_async_copy(
            tab_hbm.at[idxs.at[pl.ds(r * _CHUNK, _CHUNK)]],
            vals.at[pl.ds(r * _CHUNK, _CHUNK)], gsem).start()
        return c

    lax.fori_loop(0, _NCHUNK, gfire, 0)

    def gdrain(r, c):
        pltpu.make_async_copy(
            tab_hbm.at[idxs.at[pl.ds(0, _CHUNK)]],
            vals.at[pl.ds(0, _CHUNK)], gsem).wait()
        return c

    lax.fori_loop(0, _NCHUNK, gdrain, 0)

    wd = [wb[pl.ds(d * _L, _L)] for d in range(_ND)]

    def reduce(g, c):
        off = g * _L
        acc = jnp.zeros((_L,), jnp.float32)
        for f in range(_NS):
            acc = acc + vals[pl.ds(f * _BPW + off, _L)]
        vsum = jnp.zeros((_L,), jnp.float32)
        cnt = jnp.zeros((_L,), jnp.float32)
        for h in range(_NH):
            p = (_NS + h) * _BPW + off
            v = vals[pl.ds(p, _L)]
            ix = idxs[pl.ds(p, _L)]
            m = ix > _NS * _VOCAB
            vsum = vsum + jnp.where(m, v, 0.0)
            cnt = cnt + jnp.where(m, 1.0, 0.0)
        acc = acc + vsum / jnp.maximum(cnt, 1.0)
        for d in range(_ND):
            acc = acc + dense[pl.ds(d * _BPW + off, _L)] * wd[d]
        accs[pl.ds(off, _L)] = acc
        return c

    lax.fori_loop(0, _GPW, reduce, 0)

    pltpu.sync_copy(accs, out_hbm.at[pl.ds(base, _BPW)])


@jax.jit
def _run(sp_t, dn_t, vl_t, w_b, tab):
    mesh = plsc.VectorSubcoreMesh(core_axis_name="c", subcore_axis_name="s")
    kfn = functools.partial(
        pl.kernel,
        out_type=jax.ShapeDtypeStruct((_B,), jnp.float32),
        mesh=mesh,
        compiler_params=pltpu.CompilerParams(
            needs_layout_passes=False, use_tc_tiling_on_sc=False),
        scratch_types=[
            pltpu.VMEM((_NIDX,), jnp.int32),
            pltpu.VMEM((_NIDX,), jnp.float32),
            pltpu.VMEM((_BPW * _ND,), jnp.float32),
            pltpu.VMEM((_ND * _L,), jnp.float32),
            pltpu.VMEM((_BPW,), jnp.float32),
            pltpu.SemaphoreType.DMA,
            pltpu.SemaphoreType.DMA,
        ],
    )(_body)
    return kfn(sp_t, vl_t, dn_t, w_b, tab)


def kernel(sparse_idx, dense_vals, varlen_idx, emb_tables, W):
    foff = jnp.arange(_NS, dtype=jnp.int32) * _VOCAB
    sp_t = (sparse_idx + foff[None, :]).T.reshape(-1)
    vl_t = (varlen_idx + _NS * _VOCAB).T.reshape(-1)
    dn_t = dense_vals.T.reshape(-1)
    w_b = jnp.broadcast_to(W.reshape(_ND, 1), (_ND, _L)).reshape(-1)
    out = _run(sp_t, dn_t, vl_t, w_b, emb_tables[:, :, 0].reshape(-1))
    return out.reshape(_B, 1)


# speedup vs baseline: 4.9628x; 4.9628x over previous
import functools

import jax
import jax.numpy as jnp
from jax import lax
from jax.experimental import pallas as pl
from jax.experimental.pallas import tpu as pltpu
from jax.experimental.pallas import tpu_sc as plsc

_B = 16384
_VOCAB = 1_000_000
_NS = 26
_NSA = 13
_NSB = _NS - _NSA
_NH = 50
_ND = 13
_NT = _NS + 1
_L = 16

_NC = 2
_NSUB = 16
_NW = _NC * _NSUB
_BPW = _B // _NW
_GPW = _BPW // _L
_CHUNK = 128
_CPF = _BPW // _CHUNK

_NFA = _NSA
_NFB = _NSB + _NH
_NIDXA = _NFA * _BPW
_NIDXB = _NFB * _BPW


def _body_a(*refs):
    sp_hbm = refs[0]
    tabs = refs[1:1 + _NSA]
    out_hbm = refs[1 + _NSA]
    idxs, vals, accs, ssem, gsem = refs[2 + _NSA:]

    wid = lax.axis_index("s") * _NC + lax.axis_index("c")
    base = wid * _BPW

    def sp_stage(f, c):
        pltpu.make_async_copy(
            sp_hbm.at[pl.ds(f * _B + base, _BPW)],
            idxs.at[pl.ds(f * _BPW, _BPW)], ssem).start()
        return c

    lax.fori_loop(0, _NSA, sp_stage, 0)

    def stage_drain(f, c):
        pltpu.make_async_copy(
            sp_hbm.at[pl.ds(base, _BPW)],
            idxs.at[pl.ds(0, _BPW)], ssem).wait()
        return c

    lax.fori_loop(0, _NFA, stage_drain, 0)

    for f in range(_NSA):
        for j in range(_CPF):
            r = f * _CPF + j
            pltpu.make_async_copy(
                tabs[f].at[idxs.at[pl.ds(r * _CHUNK, _CHUNK)]],
                vals.at[pl.ds(r * _CHUNK, _CHUNK)], gsem).start()

    def gdrain(r, c):
        pltpu.make_async_copy(
            tabs[0].at[idxs.at[pl.ds(0, _CHUNK)]],
            vals.at[pl.ds(0, _CHUNK)], gsem).wait()
        return c

    lax.fori_loop(0, _NFA * _CPF, gdrain, 0)

    def reduce(g, c):
        off = g * _L
        acc = jnp.zeros((_L,), jnp.float32)
        for f in range(_NSA):
            acc = acc + vals[pl.ds(f * _BPW + off, _L)]
        accs[pl.ds(off, _L)] = acc
        return c

    lax.fori_loop(0, _GPW, reduce, 0)

    pltpu.sync_copy(accs, out_hbm.at[pl.ds(base, _BPW)])


def _body_b(*refs):
    sp_hbm, vl_hbm, dn_hbm, wb_hbm, pt_hbm = refs[:5]
    tabs = refs[5:5 + _NSB + 1]
    out_hbm = refs[6 + _NSB]
    idxs, vals, dense, wb, part, accs, ssem, gsem = refs[7 + _NSB:]

    wid = lax.axis_index("s") * _NC + lax.axis_index("c")
    base = wid * _BPW

    def sp_stage(f, c):
        pltpu.make_async_copy(
            sp_hbm.at[pl.ds((_NSA + f) * _B + base, _BPW)],
            idxs.at[pl.ds(f * _BPW, _BPW)], ssem).start()
        return c

    def vl_stage(h, c):
        pltpu.make_async_copy(
            vl_hbm.at[pl.ds(h * _B + base, _BPW)],
            idxs.at[pl.ds((_NSB + h) * _BPW, _BPW)], ssem).start()
        return c

    def dn_stage(d, c):
        pltpu.make_async_copy(
            dn_hbm.at[pl.ds(d * _B + base, _BPW)],
            dense.at[pl.ds(d * _BPW, _BPW)], ssem).start()
        return c

    lax.fori_loop(0, _NSB, sp_stage, 0)
    lax.fori_loop(0, _NH, vl_stage, 0)
    lax.fori_loop(0, _ND, dn_stage, 0)
    pltpu.make_async_copy(
        pt_hbm.at[pl.ds(base, _BPW)], part, ssem).start()
    pltpu.sync_copy(wb_hbm, wb)

    def stage_drain(f, c):
        pltpu.make_async_copy(
            sp_hbm.at[pl.ds(base, _BPW)],
            idxs.at[pl.ds(0, _BPW)], ssem).wait()
        return c

    lax.fori_loop(0, _NFB + _ND + 1, stage_drain, 0)

    for f in range(_NSB):
        for j in range(_CPF):
            r = f * _CPF + j
            pltpu.make_async_copy(
                tabs[f].at[idxs.at[pl.ds(r * _CHUNK, _CHUNK)]],
                vals.at[pl.ds(r * _CHUNK, _CHUNK)], gsem).start()

    def vfire(t, c):
        r = _NSB * _CPF + t
        pltpu.make_async_copy(
            tabs[_NSB].at[idxs.at[pl.ds(r * _CHUNK, _CHUNK)]],
            vals.at[pl.ds(r * _CHUNK, _CHUNK)], gsem).start()
        return c

    lax.fori_loop(0, _NH * _CPF, vfire, 0)

    def gdrain(r, c):
        pltpu.make_async_copy(
            tabs[0].at[idxs.at[pl.ds(0, _CHUNK)]],
            vals.at[pl.ds(0, _CHUNK)], gsem).wait()
        return c

    lax.fori_loop(0, _NFB * _CPF, gdrain, 0)

    wd = [wb[pl.ds(d * _L, _L)] for d in range(_ND)]

    def reduce(g, c):
        off = g * _L
        acc = part[pl.ds(off, _L)]
        for f in range(_NSB):
            acc = acc + vals[pl.ds(f * _BPW + off, _L)]
        vsum = jnp.zeros((_L,), jnp.float32)
        cnt = jnp.zeros((_L,), jnp.float32)
        for h in range(_NH):
            p = (_NSB + h) * _BPW + off
            v = vals[pl.ds(p, _L)]
            ix = idxs[pl.ds(p, _L)]
            m = ix > 0
            vsum = vsum + jnp.where(m, v, 0.0)
            cnt = cnt + jnp.where(m, 1.0, 0.0)
        acc = acc + vsum / jnp.maximum(cnt, 1.0)
        for d in range(_ND):
            acc = acc + dense[pl.ds(d * _BPW + off, _L)] * wd[d]
        accs[pl.ds(off, _L)] = acc
        return c

    lax.fori_loop(0, _GPW, reduce, 0)

    pltpu.sync_copy(accs, out_hbm.at[pl.ds(base, _BPW)])


@jax.jit
def _run(sp_t, dn_t, vl_t, w_b, *tabs):
    mesh = plsc.VectorSubcoreMesh(core_axis_name="c", subcore_axis_name="s")
    params = pltpu.CompilerParams(
        needs_layout_passes=False, use_tc_tiling_on_sc=False)
    kfn_a = functools.partial(
        pl.kernel,
        out_type=jax.ShapeDtypeStruct((_B,), jnp.float32),
        mesh=mesh,
        compiler_params=params,
        scratch_types=[
            pltpu.VMEM((_NIDXA,), jnp.int32),
            pltpu.VMEM((_NIDXA,), jnp.float32),
            pltpu.VMEM((_BPW,), jnp.float32),
            pltpu.SemaphoreType.DMA,
            pltpu.SemaphoreType.DMA,
        ],
    )(_body_a)
    kfn_b = functools.partial(
        pl.kernel,
        out_type=jax.ShapeDtypeStruct((_B,), jnp.float32),
        mesh=mesh,
        compiler_params=params,
        scratch_types=[
            pltpu.VMEM((_NIDXB,), jnp.int32),
            pltpu.VMEM((_NIDXB,), jnp.float32),
            pltpu.VMEM((_BPW * _ND,), jnp.float32),
            pltpu.VMEM((_ND * _L,), jnp.float32),
            pltpu.VMEM((_BPW,), jnp.float32),
            pltpu.VMEM((_BPW,), jnp.float32),
            pltpu.SemaphoreType.DMA,
            pltpu.SemaphoreType.DMA,
        ],
    )(_body_b)
    part = kfn_a(sp_t, *tabs[:_NSA])
    return kfn_b(sp_t, vl_t, dn_t, w_b, part, *tabs[_NSA:])


def kernel(sparse_idx, dense_vals, varlen_idx, emb_tables, W):
    sp_t = sparse_idx.T.reshape(-1)
    vl_t = varlen_idx.T.reshape(-1)
    dn_t = dense_vals.T.reshape(-1)
    tabs = [emb_tables[f, :, 0] for f in range(_NT)]
    w_b = jnp.broadcast_to(W.reshape(_ND, 1), (_ND, _L)).reshape(-1)
    out = _run(sp_t, dn_t, vl_t, w_b, *tabs)
    return out.reshape(_B, 1)
